# baseline (speedup 1.0000x reference)
import functools
import jax
import jax.numpy as jnp
from jax import lax
from jax.experimental import pallas as pl
from jax.experimental.pallas import tpu as pltpu
from jax.experimental.pallas import tpu_sc as plsc

_P = 0.1
_CHUNK = 16
_HALF = 256


def _delete_mask_const(B, L, dtype=jnp.float32):
    rand = jax.random.uniform(jax.random.key(42), (B, L))
    return (rand < _P).astype(dtype)


def _mask_kernel(msg_ref, rand_ref, idx_ref):
    b = pl.program_id(0)
    m = msg_ref[0]
    L, V = m.shape
    f32, i32 = jnp.float32, jnp.int32

    col0 = m[:, 0:1]
    rmax = jnp.max(m, axis=1, keepdims=True)
    nz_col = (rmax > col0).astype(f32)

    iota_col = lax.broadcasted_iota(i32, (L, 1), 0).astype(f32)
    iota_row = lax.broadcasted_iota(i32, (1, L), 1).astype(f32)
    eye = (lax.broadcasted_iota(i32, (L, L), 0) ==
           lax.broadcasted_iota(i32, (L, L), 1)).astype(f32)

    nz_row = lax.dot_general(nz_col, eye, (((0,), (0,)), ((), ())),
                             preferred_element_type=f32)
    randlt = rand_ref[0]
    keep_row = 1.0 - nz_row * randlt

    tri = (lax.broadcasted_iota(i32, (L, L), 0) <=
           lax.broadcasted_iota(i32, (L, L), 1)).astype(f32)
    prefix = jnp.dot(keep_row, tri, preferred_element_type=f32)
    kc = jnp.sum(keep_row)
    dest = prefix - 1.0

    sel = (iota_col == dest).astype(f32) * keep_row
    src = lax.dot_general(iota_row, sel, (((1,), (1,)), ((), ())),
                          preferred_element_type=f32)
    pad = (iota_row >= kc).astype(f32)
    idx_ref[0] = (src + pad * 65536.0).astype(i32) + b * L


def _sc_gather_body(table, idx_hbm, eos_hbm, out_hbm,
                    idx_v, rows_v, eos_v, sem):
    L = 512
    i32 = jnp.int32
    c = lax.axis_index("c")
    s = lax.axis_index("s")
    wid = s * 2 + c
    b = wid // 2
    half = wid % 2
    g0 = b * L + half * _HALF

    pltpu.sync_copy(idx_hbm.at[pl.ds(g0, _HALF)], idx_v)
    pltpu.sync_copy(eos_hbm, eos_v)

    lanes = lax.broadcasted_iota(i32, (_CHUNK,), 0)
    for ci in range(_HALF // _CHUNK):
        iv_raw = idx_v[pl.ds(ci * _CHUNK, _CHUNK)]
        eosmask = iv_raw >= 65536
        iv = jnp.bitwise_and(iv_raw, 65535)
        full_eos = jnp.all(eosmask)
        any_eos = jnp.any(eosmask)
        dst = out_hbm.at[pl.ds(g0 + ci * _CHUNK, _CHUNK)]

        @pl.when(jnp.logical_not(full_eos))
        def _():
            pltpu.async_copy(table.at[iv], rows_v, sem).wait()

        @pl.when(jnp.logical_not(any_eos))
        def _():
            pltpu.sync_copy(rows_v, dst)

        @pl.when(full_eos)
        def _():
            pltpu.sync_copy(eos_v, dst)

        @pl.when(jnp.logical_and(any_eos, jnp.logical_not(full_eos)))
        def _():
            def row_body(r, carry):
                dr = out_hbm.at[pl.ds(g0 + ci * _CHUNK + r, 1)]
                is_eos = jnp.any(jnp.logical_and(eosmask, lanes == r))

                @pl.when(jnp.logical_not(is_eos))
                def _():
                    pltpu.sync_copy(rows_v.at[pl.ds(r, 1)], dr)

                @pl.when(is_eos)
                def _():
                    pltpu.sync_copy(eos_v.at[pl.ds(r, 1)], dr)

                return carry
            lax.fori_loop(0, _CHUNK, row_body, 0)


def kernel(message, message_length, apply_noise):
    del message_length
    B, L, V = message.shape
    f32, i32 = jnp.float32, jnp.int32
    randlt = _delete_mask_const(B, L).reshape(B, 1, L)

    idx_out = pl.pallas_call(
        _mask_kernel,
        grid=(B,),
        in_specs=[
            pl.BlockSpec((1, L, V), lambda b: (b, 0, 0)),
            pl.BlockSpec((1, 1, L), lambda b: (b, 0, 0)),
        ],
        out_specs=pl.BlockSpec((1, 1, L), lambda b: (b, 0, 0)),
        out_shape=jax.ShapeDtypeStruct((B, 1, L), i32),
        compiler_params=pltpu.CompilerParams(
            dimension_semantics=("arbitrary",),
        ),
    )(message, randlt)

    idx_flat = idx_out.reshape(B * L)
    table = message.reshape(B * L, V)
    eos = jnp.zeros((_CHUNK, V), f32).at[:, 0].set(1.0)

    mesh = plsc.VectorSubcoreMesh(core_axis_name="c", subcore_axis_name="s")
    sc_call = functools.partial(
        pl.kernel, mesh=mesh,
        compiler_params=pltpu.CompilerParams(needs_layout_passes=False),
        out_type=jax.ShapeDtypeStruct((B * L, V), f32),
        scratch_types=[
            pltpu.VMEM((_HALF,), i32),
            pltpu.VMEM((_CHUNK, V), f32),
            pltpu.VMEM((_CHUNK, V), f32),
            pltpu.SemaphoreType.DMA,
        ],
    )
    out_flat = sc_call(_sc_gather_body)(table, idx_flat, eos)

    out = out_flat.reshape(B, L, V)
    return jnp.where(jnp.asarray(apply_noise) != 0, out, message)


# speedup vs baseline: 1.0366x; 1.0366x over previous
import functools
import jax
import jax.numpy as jnp
from jax import lax
from jax.experimental import pallas as pl
from jax.experimental.pallas import tpu as pltpu
from jax.experimental.pallas import tpu_sc as plsc

_P = 0.1
_CHUNK = 16
_HALF = 256


def _delete_mask_const(B, L, dtype=jnp.float32):
    rand = jax.random.uniform(jax.random.key(42), (B, L))
    return (rand < _P).astype(dtype)


def _mask_kernel(msg_ref, rand_ref, idx_ref):
    b = pl.program_id(0)
    m = msg_ref[0]
    L, V = m.shape
    f32, i32 = jnp.float32, jnp.int32

    col0 = m[:, 0:1]
    rmax = jnp.max(m, axis=1, keepdims=True)
    nz_col = (rmax > col0).astype(f32)

    iota_col = lax.broadcasted_iota(i32, (L, 1), 0).astype(f32)
    iota_row = lax.broadcasted_iota(i32, (1, L), 1).astype(f32)
    eye = (lax.broadcasted_iota(i32, (L, L), 0) ==
           lax.broadcasted_iota(i32, (L, L), 1)).astype(f32)

    nz_row = lax.dot_general(nz_col, eye, (((0,), (0,)), ((), ())),
                             preferred_element_type=f32)
    randlt = rand_ref[0]
    keep_row = 1.0 - nz_row * randlt

    tri = (lax.broadcasted_iota(i32, (L, L), 0) <=
           lax.broadcasted_iota(i32, (L, L), 1)).astype(f32)
    prefix = jnp.dot(keep_row, tri, preferred_element_type=f32)
    kc = jnp.sum(keep_row)
    dest = prefix - 1.0

    sel = (iota_col == dest).astype(f32) * keep_row
    src = lax.dot_general(iota_row, sel, (((1,), (1,)), ((), ())),
                          preferred_element_type=f32)
    pad = (iota_row >= kc).astype(f32)
    idx_ref[0] = (src + pad * 65536.0).astype(i32) + b * L


def _sc_gather_body(table, idx_hbm, eos_hbm, out_hbm,
                    idx_v, rows_v, eos_v, sem0, sem1):
    L = 512
    i32 = jnp.int32
    c = lax.axis_index("c")
    s = lax.axis_index("s")
    wid = s * 2 + c
    b = wid // 2
    half = wid % 2
    g0 = b * L + half * _HALF

    pltpu.sync_copy(idx_hbm.at[pl.ds(g0, _HALF)], idx_v)
    pltpu.sync_copy(eos_hbm, eos_v)

    lanes = lax.broadcasted_iota(i32, (_CHUNK,), 0)
    nchunks = _HALF // _CHUNK
    sems = (sem0, sem1)

    def start_gather(ci):
        iv_raw = idx_v[pl.ds(ci * _CHUNK, _CHUNK)]
        eosmask = iv_raw >= 65536
        iv = jnp.bitwise_and(iv_raw, 65535)
        cp = pltpu.async_copy(table.at[iv], rows_v.at[ci % 2], sems[ci % 2])
        return cp, eosmask

    pending = {0: start_gather(0)}
    for ci in range(nchunks):
        if ci + 1 < nchunks:
            pending[ci + 1] = start_gather(ci + 1)
        cp, eosmask = pending.pop(ci)
        cp.wait()
        buf = rows_v.at[ci % 2]
        full_eos = jnp.all(eosmask)
        any_eos = jnp.any(eosmask)
        dst = out_hbm.at[pl.ds(g0 + ci * _CHUNK, _CHUNK)]

        @pl.when(jnp.logical_not(any_eos))
        def _():
            pltpu.sync_copy(buf, dst)

        @pl.when(full_eos)
        def _():
            pltpu.sync_copy(eos_v, dst)

        @pl.when(jnp.logical_and(any_eos, jnp.logical_not(full_eos)))
        def _():
            def row_body(r, carry):
                dr = out_hbm.at[pl.ds(g0 + ci * _CHUNK + r, 1)]
                is_eos = jnp.any(jnp.logical_and(eosmask, lanes == r))

                @pl.when(jnp.logical_not(is_eos))
                def _():
                    pltpu.sync_copy(buf.at[pl.ds(r, 1)], dr)

                @pl.when(is_eos)
                def _():
                    pltpu.sync_copy(eos_v.at[pl.ds(r, 1)], dr)

                return carry
            lax.fori_loop(0, _CHUNK, row_body, 0)


def kernel(message, message_length, apply_noise):
    del message_length
    B, L, V = message.shape
    f32, i32 = jnp.float32, jnp.int32
    randlt = _delete_mask_const(B, L).reshape(B, 1, L)

    idx_out = pl.pallas_call(
        _mask_kernel,
        grid=(B,),
        in_specs=[
            pl.BlockSpec((1, L, V), lambda b: (b, 0, 0)),
            pl.BlockSpec((1, 1, L), lambda b: (b, 0, 0)),
        ],
        out_specs=pl.BlockSpec((1, 1, L), lambda b: (b, 0, 0)),
        out_shape=jax.ShapeDtypeStruct((B, 1, L), i32),
        compiler_params=pltpu.CompilerParams(
            dimension_semantics=("arbitrary",),
        ),
    )(message, randlt)

    idx_flat = idx_out.reshape(B * L)
    table = message.reshape(B * L, V)
    eos = jnp.zeros((_CHUNK, V), f32).at[:, 0].set(1.0)

    mesh = plsc.VectorSubcoreMesh(core_axis_name="c", subcore_axis_name="s")
    sc_call = functools.partial(
        pl.kernel, mesh=mesh,
        compiler_params=pltpu.CompilerParams(needs_layout_passes=False),
        out_type=jax.ShapeDtypeStruct((B * L, V), f32),
        scratch_types=[
            pltpu.VMEM((_HALF,), i32),
            pltpu.VMEM((2, _CHUNK, V), f32),
            pltpu.VMEM((_CHUNK, V), f32),
            pltpu.SemaphoreType.DMA,
            pltpu.SemaphoreType.DMA,
        ],
    )
    out_flat = sc_call(_sc_gather_body)(table, idx_flat, eos)

    out = out_flat.reshape(B, L, V)
    return jnp.where(jnp.asarray(apply_noise) != 0, out, message)
